# baseline (speedup 1.0000x reference)
import jax
import jax.numpy as jnp
from jax.experimental import pallas as pl
from jax.experimental.pallas import tpu as pltpu

_N, _D, _H, _B = 16384, 512, 256, 16
_BLK = 2048
_NB = _N // _BLK
_NEG = -1e30


def _pool_body(xb, bb, w1t, w2, out,
               acc, mstate, sstate, accf, ef, xprev, eprev):
    i = pl.program_id(0)

    @pl.when(i == 0)
    def _init():
        acc[:] = jnp.zeros_like(acc)
        mstate[:] = jnp.full_like(mstate, _NEG)
        sstate[:] = jnp.zeros_like(sstate)
        accf[:] = jnp.full_like(accf, _NEG)
        ef[:] = jnp.full_like(ef, _NEG)
        xprev[:] = jnp.zeros_like(xprev)
        eprev[:] = jnp.zeros_like(eprev)

    x_bf = xb[:].astype(jnp.bfloat16)

    t = jnp.tanh(jnp.dot(x_bf, w1t[:], preferred_element_type=jnp.float32))
    a = jax.lax.dot_general(w2[:], t.astype(jnp.bfloat16),
                            (((1,), (1,)), ((), ())),
                            preferred_element_type=jnp.float32)
    b = bb[0]
    seg = jax.lax.broadcasted_iota(jnp.int32, (_B, _BLK), 0)
    mask = b == seg
    am = jnp.where(mask, a, _NEG)
    m_blk = jnp.max(am, axis=1, keepdims=True)
    m_old = mstate[:]
    m_new = jnp.maximum(m_old, m_blk)
    scale = jnp.exp(m_old - m_new)
    maskf = mask.astype(jnp.float32)
    m_tok = jnp.sum(maskf * m_new, axis=0, keepdims=True)
    e_row = jnp.exp(a - m_tok)
    e_mat = maskf * e_row
    sstate[:] = sstate[:] * scale + jnp.sum(e_mat, axis=1, keepdims=True)
    mstate[:] = m_new

    @pl.when(i > 0)
    def _pool_prev():
        acc[:] = acc[:] * jnp.exp(accf[:] - ef[:]) + jax.lax.dot_general(
            eprev[:], xprev[:], (((1,), (0,)), ((), ())),
            preferred_element_type=jnp.float32)
        accf[:] = ef[:]

    xprev[:] = x_bf
    eprev[:] = e_mat.astype(jnp.bfloat16)
    ef[:] = m_new

    @pl.when(i == _NB - 1)
    def _fin():
        z = acc[:] * jnp.exp(accf[:] - m_new) + jax.lax.dot_general(
            e_mat.astype(jnp.bfloat16), x_bf, (((1,), (0,)), ((), ())),
            preferred_element_type=jnp.float32)
        out[:] = z / (sstate[:] + 1e-16)


def kernel(x, batch, W1, W2):
    batch3 = batch.astype(jnp.int32).reshape(_NB, 1, _BLK)
    w1t = W1.T.astype(jnp.bfloat16)
    w2 = W2.astype(jnp.bfloat16)
    return pl.pallas_call(
        _pool_body,
        grid=(_NB,),
        in_specs=[
            pl.BlockSpec((_BLK, _D), lambda i: (i, 0)),
            pl.BlockSpec((1, 1, _BLK), lambda i: (i, 0, 0)),
            pl.BlockSpec((_D, _H), lambda i: (0, 0)),
            pl.BlockSpec((1, _H), lambda i: (0, 0)),
        ],
        out_specs=pl.BlockSpec((_B, _D), lambda i: (0, 0)),
        out_shape=jax.ShapeDtypeStruct((_B, _D), jnp.float32),
        scratch_shapes=[
            pltpu.VMEM((_B, _D), jnp.float32),
            pltpu.VMEM((_B, 1), jnp.float32),
            pltpu.VMEM((_B, 1), jnp.float32),
            pltpu.VMEM((_B, 1), jnp.float32),
            pltpu.VMEM((_B, 1), jnp.float32),
            pltpu.VMEM((_BLK, _D), jnp.bfloat16),
            pltpu.VMEM((_B, _BLK), jnp.bfloat16),
        ],
    )(x, batch3, w1t, w2)


# speedup vs baseline: 1.0986x; 1.0986x over previous
import jax
import jax.numpy as jnp
from jax.experimental import pallas as pl
from jax.experimental.pallas import tpu as pltpu

_N, _D, _H, _B = 16384, 512, 256, 16
_BLK = 4096
_NB = _N // _BLK
_DA = _D + 128


def _pool_body(xb, bb, w1t, w2t, out, acc, mprev, aprev):
    i = pl.program_id(0)

    @pl.when(i == 0)
    def _init():
        acc[:] = jnp.zeros_like(acc)

    x_bf = xb[:].astype(jnp.bfloat16)
    t = jnp.tanh(jnp.dot(x_bf, w1t[:], preferred_element_type=jnp.float32))
    a = jnp.dot(t.astype(jnp.bfloat16), w2t[:],
                preferred_element_type=jnp.float32)
    e_bf = jnp.exp(a).astype(jnp.bfloat16)
    x_e = x_bf * e_bf
    e_wide = jnp.broadcast_to(e_bf, (_BLK, 128))
    x_aug = jnp.concatenate([x_e, e_wide], axis=1)
    b = bb[0]
    seg = jax.lax.broadcasted_iota(jnp.int32, (_B, _BLK), 0)
    maskf = (b == seg).astype(jnp.bfloat16)

    @pl.when(i > 0)
    def _pool_prev():
        acc[:] = acc[:] + jax.lax.dot_general(
            mprev[:], aprev[:], (((1,), (0,)), ((), ())),
            preferred_element_type=jnp.float32)

    mprev[:] = maskf
    aprev[:] = x_aug

    @pl.when(i == _NB - 1)
    def _fin():
        z = acc[:] + jax.lax.dot_general(
            maskf, x_aug, (((1,), (0,)), ((), ())),
            preferred_element_type=jnp.float32)
        out[:] = z[:, :_D] / (z[:, _D:_D + 1] + 1e-16)


def kernel(x, batch, W1, W2):
    batch3 = batch.astype(jnp.int32).reshape(_NB, 1, _BLK)
    w1t = W1.T.astype(jnp.bfloat16)
    w2t = W2.T.astype(jnp.bfloat16)
    return pl.pallas_call(
        _pool_body,
        grid=(_NB,),
        in_specs=[
            pl.BlockSpec((_BLK, _D), lambda i: (i, 0)),
            pl.BlockSpec((1, 1, _BLK), lambda i: (i, 0, 0)),
            pl.BlockSpec((_D, _H), lambda i: (0, 0)),
            pl.BlockSpec((_H, 1), lambda i: (0, 0)),
        ],
        out_specs=pl.BlockSpec((_B, _D), lambda i: (0, 0)),
        out_shape=jax.ShapeDtypeStruct((_B, _D), jnp.float32),
        scratch_shapes=[
            pltpu.VMEM((_B, _DA), jnp.float32),
            pltpu.VMEM((_B, _BLK), jnp.bfloat16),
            pltpu.VMEM((_BLK, _DA), jnp.bfloat16),
        ],
    )(x, batch3, w1t, w2t)
